# baseline (speedup 1.0000x reference)
import functools

import jax
import jax.numpy as jnp
from jax import lax
from jax.experimental import pallas as pl
from jax.experimental.pallas import tpu as pltpu
from jax.experimental.pallas import tpu_sc as plsc

N = 10000
E = 320000
D = 128

NC = 2
NS = 16
NW = NC * NS
CH = 128
G = 80
E_pad = NW * G * CH
NPT = 640
N_pad = NS * NPT

_mesh = plsc.VectorSubcoreMesh(core_axis_name="c", subcore_axis_name="s")


@functools.partial(
    pl.kernel,
    out_type=jax.ShapeDtypeStruct((NC, 2, N_pad, D), jnp.float32),
    mesh=_mesh,
    scratch_types=[
        pltpu.VMEM((2 * G, CH), jnp.int32),
        pltpu.VMEM((CH, D), jnp.float32),
        pltpu.VMEM_SHARED((N_pad, D), jnp.float32),
        pltpu.SemaphoreType.DMA,
    ],
)
def _sc_degrees(ei, ones_h, zeros_h, out, idx_v, ones_v, acc, isem):
    c = lax.axis_index("c")
    s = lax.axis_index("s")
    w = s * NC + c
    r0 = s * NPT
    cp = pltpu.async_copy(ei.at[w], idx_v, isem)
    pltpu.sync_copy(ones_h, ones_v)
    cp.wait()
    for which in range(2):
        for i in range(NPT // CH):
            pltpu.sync_copy(zeros_h, acc.at[pl.ds(r0 + i * CH, CH)])
        plsc.subcore_barrier()

        def body(g, carry):
            pltpu.sync_copy(ones_v, acc.at[idx_v.at[2 * g + which]], add=True)
            return carry

        lax.fori_loop(0, G, body, 0)
        plsc.subcore_barrier()
        pltpu.sync_copy(acc.at[pl.ds(r0, NPT)], out.at[c, which, pl.ds(r0, NPT)])
        plsc.subcore_barrier()


@functools.partial(
    pl.kernel,
    out_type=jax.ShapeDtypeStruct((NC, N_pad, D), jnp.float32),
    mesh=_mesh,
    scratch_types=[
        pltpu.VMEM((4, 2, CH), jnp.int32),
        pltpu.VMEM((CH, D), jnp.float32),
        pltpu.VMEM((CH, D), jnp.float32),
        pltpu.VMEM_SHARED((N_pad, D), jnp.float32),
        pltpu.SemaphoreType.DMA,
        pltpu.SemaphoreType.DMA,
        pltpu.SemaphoreType.DMA,
        pltpu.SemaphoreType.DMA,
        pltpu.SemaphoreType.DMA,
        pltpu.SemaphoreType.DMA,
    ],
)
def _sc_aggregate(y, ei, zeros_h, out, idxb, rows0, rows1, acc,
                  isem0, isem1, isem2, isem3, gsem0, gsem1):
    c = lax.axis_index("c")
    s = lax.axis_index("s")
    w = s * NC + c
    r0 = s * NPT
    isems = [isem0, isem1, isem2, isem3]
    rows = [rows0, rows1]
    gsems = [gsem0, gsem1]

    def idx_start(slot, gnum):
        pltpu.async_copy(ei.at[w, gnum], idxb.at[slot], isems[slot])

    def idx_wait(slot):
        pltpu.make_async_copy(ei.at[w, 0], idxb.at[slot], isems[slot]).wait()

    def gather_start(slot, p):
        pltpu.async_copy(y.at[idxb.at[slot, 0]], rows[p], gsems[p])

    def gather_wait(p):
        pltpu.make_async_copy(y.at[pl.ds(0, CH)], rows[p], gsems[p]).wait()

    for i in range(NPT // CH):
        pltpu.sync_copy(zeros_h, acc.at[pl.ds(r0 + i * CH, CH)])
    plsc.subcore_barrier()

    for k in range(4):
        idx_start(k, k)
    idx_wait(0)
    gather_start(0, 0)
    idx_wait(1)
    gather_start(1, 1)

    def body(i, carry):
        base = 4 * i
        for k in range(4):
            p = k % 2
            gather_wait(p)
            pltpu.sync_copy(rows[p], acc.at[idxb.at[k, 1]], add=True)
            idx_start(k, jnp.minimum(base + k + 4, G - 1))
            k2 = (k + 2) % 4
            idx_wait(k2)
            gather_start(k2, p)
        return carry

    lax.fori_loop(0, G // 4, body, 0)
    idx_wait(2)
    idx_wait(3)
    gather_wait(0)
    gather_wait(1)
    plsc.subcore_barrier()
    pltpu.sync_copy(acc.at[pl.ds(r0, NPT)], out.at[c, pl.ds(r0, NPT)])


def _tc_norms_body(deg_ref, ns_ref, nd_ref):
    ds = deg_ref[0, 0, :, 0:1] + deg_ref[1, 0, :, 0:1]
    dd = deg_ref[0, 1, :, 0:1] + deg_ref[1, 1, :, 0:1]
    ns_ref[...] = lax.rsqrt(jnp.maximum(ds, 1.0))
    nd_ref[...] = lax.rsqrt(jnp.maximum(dd, 1.0))


def _tc_matmul_body(x_ref, w_ref, z_ref):
    z_ref[...] = jnp.dot(x_ref[...], w_ref[...],
                         preferred_element_type=jnp.float32)


def _tc_scale_body(z_ref, ns_ref, y_ref):
    y_ref[...] = z_ref[...] * ns_ref[...][:N]


def _tc_mid_body(p_ref, nd_ref, ns_ref, b_ref, w_ref, y_ref):
    h = (p_ref[0] + p_ref[1]) * nd_ref[...] + b_ref[...]
    h = jnp.maximum(h, 0.0) * ns_ref[...]
    y_ref[...] = jnp.dot(h, w_ref[...], preferred_element_type=jnp.float32)


def _tc_final_body(p_ref, nd_ref, b_ref, wout_ref, bout_ref, o_ref):
    nd = nd_ref[...][:N]
    h = (p_ref[0, :N] + p_ref[1, :N]) * nd + b_ref[...]
    m = jnp.sum(h, axis=0, keepdims=True) * (1.0 / N)
    o_ref[...] = jnp.dot(m, wout_ref[...],
                         preferred_element_type=jnp.float32) + bout_ref[...]


def _tc_norms(deg):
    return pl.pallas_call(
        _tc_norms_body,
        out_shape=(jax.ShapeDtypeStruct((N_pad, 1), jnp.float32),
                   jax.ShapeDtypeStruct((N_pad, 1), jnp.float32)),
    )(deg)


def _tc_matmul(x, W):
    return pl.pallas_call(
        _tc_matmul_body,
        out_shape=jax.ShapeDtypeStruct((N, D), jnp.float32),
    )(x, W)


def _tc_scale(z, ns):
    return pl.pallas_call(
        _tc_scale_body,
        out_shape=jax.ShapeDtypeStruct((N, D), jnp.float32),
    )(z, ns)


def _tc_mid(p, nd, ns, b, W):
    return pl.pallas_call(
        _tc_mid_body,
        out_shape=jax.ShapeDtypeStruct((N_pad, D), jnp.float32),
    )(p, nd, ns, b, W)


def _tc_final(p, nd, b, Wout, bout):
    return pl.pallas_call(
        _tc_final_body,
        out_shape=jax.ShapeDtypeStruct((1, 1), jnp.float32),
    )(p, nd, b, Wout, bout)


def kernel(in_feat, edge_index, W1, b1, W2, b2, W3, b3, Wout, bout):
    src = edge_index[0]
    dst = edge_index[1]
    pad = E_pad - E
    sink = N + (jnp.arange(pad, dtype=jnp.int32) % (N_pad - N))
    spread = jnp.arange(pad, dtype=jnp.int32) % N
    src_a = jnp.concatenate([src, spread])
    dst_a = jnp.concatenate([dst, sink])
    src_d = jnp.concatenate([src, sink])
    ei = jnp.stack([src_a.reshape(NW, G, CH), dst_a.reshape(NW, G, CH)], axis=2)
    ei_d = jnp.stack(
        [src_d.reshape(NW, G, CH), dst_a.reshape(NW, G, CH)], axis=2
    ).reshape(NW, 2 * G, CH)

    ones_h = jnp.ones((CH, D), jnp.float32)
    zerosD = jnp.zeros((CH, D), jnp.float32)

    z = _tc_matmul(in_feat, W1)
    deg = _sc_degrees(ei_d, ones_h, zerosD)
    ns, nd = _tc_norms(deg)

    b1r = b1.reshape(1, D)
    b2r = b2.reshape(1, D)
    b3r = b3.reshape(1, D)

    y = _tc_scale(z, ns)
    p = _sc_aggregate(y, ei, zerosD)
    y = _tc_mid(p, nd, ns, b1r, W2)
    p = _sc_aggregate(y, ei, zerosD)
    y = _tc_mid(p, nd, ns, b2r, W2)
    p = _sc_aggregate(y, ei, zerosD)
    y = _tc_mid(p, nd, ns, b2r, W3)
    p = _sc_aggregate(y, ei, zerosD)
    out = _tc_final(p, nd, b3r, Wout, bout.reshape(1, 1))
    return out.reshape(1)


# speedup vs baseline: 1.0008x; 1.0008x over previous
import functools

import jax
import jax.numpy as jnp
from jax import lax
from jax.experimental import pallas as pl
from jax.experimental.pallas import tpu as pltpu
from jax.experimental.pallas import tpu_sc as plsc

N = 10000
E = 320000
D = 128

NC = 2
NS = 16
NW = NC * NS
CH = 128
G = 80
E_pad = NW * G * CH
NPT = 640
N_pad = NS * NPT

_mesh = plsc.VectorSubcoreMesh(core_axis_name="c", subcore_axis_name="s")


@functools.partial(
    pl.kernel,
    out_type=jax.ShapeDtypeStruct((NC, 2, N_pad, D), jnp.float32),
    mesh=_mesh,
    scratch_types=[
        pltpu.VMEM((2 * G, CH), jnp.int32),
        pltpu.VMEM((CH, D), jnp.float32),
        pltpu.VMEM_SHARED((N_pad, D), jnp.float32),
        pltpu.SemaphoreType.DMA,
    ],
)
def _sc_degrees(ei, ones_h, zeros_h, out, idx_v, ones_v, acc, isem):
    c = lax.axis_index("c")
    s = lax.axis_index("s")
    w = s * NC + c
    r0 = s * NPT
    cp = pltpu.async_copy(ei.at[w], idx_v, isem)
    pltpu.sync_copy(ones_h, ones_v)
    cp.wait()
    for which in range(2):
        for i in range(NPT // CH):
            pltpu.sync_copy(zeros_h, acc.at[pl.ds(r0 + i * CH, CH)])
        plsc.subcore_barrier()

        def body(g, carry):
            pltpu.sync_copy(ones_v, acc.at[idx_v.at[2 * g + which]], add=True)
            return carry

        lax.fori_loop(0, G, body, 0)
        plsc.subcore_barrier()
        pltpu.sync_copy(acc.at[pl.ds(r0, NPT)], out.at[c, which, pl.ds(r0, NPT)])
        plsc.subcore_barrier()


@functools.partial(
    pl.kernel,
    out_type=jax.ShapeDtypeStruct((NC, N_pad, D), jnp.float32),
    mesh=_mesh,
    scratch_types=[
        pltpu.VMEM((4, 2, CH), jnp.int32),
        pltpu.VMEM((CH, D), jnp.float32),
        pltpu.VMEM((CH, D), jnp.float32),
        pltpu.VMEM_SHARED((N_pad, D), jnp.float32),
        pltpu.SemaphoreType.DMA,
        pltpu.SemaphoreType.DMA,
        pltpu.SemaphoreType.DMA,
        pltpu.SemaphoreType.DMA,
        pltpu.SemaphoreType.DMA,
        pltpu.SemaphoreType.DMA,
    ],
)
def _sc_aggregate(y, ei, zeros_h, out, idxb, rows0, rows1, acc,
                  isem0, isem1, isem2, isem3, gsem0, gsem1):
    c = lax.axis_index("c")
    s = lax.axis_index("s")
    w = s * NC + c
    r0 = s * NPT
    isems = [isem0, isem1, isem2, isem3]
    rows = [rows0, rows1]
    gsems = [gsem0, gsem1]

    def idx_start(slot, gnum):
        pltpu.async_copy(ei.at[w, gnum], idxb.at[slot], isems[slot])

    def idx_wait(slot):
        pltpu.make_async_copy(ei.at[w, 0], idxb.at[slot], isems[slot]).wait()

    def gather_start(slot, p):
        pltpu.async_copy(y.at[idxb.at[slot, 0]], rows[p], gsems[p])

    def gather_wait(p):
        pltpu.make_async_copy(y.at[pl.ds(0, CH)], rows[p], gsems[p]).wait()

    for i in range(NPT // CH):
        pltpu.sync_copy(zeros_h, acc.at[pl.ds(r0 + i * CH, CH)])
    plsc.subcore_barrier()

    for k in range(4):
        idx_start(k, k)
    idx_wait(0)
    gather_start(0, 0)
    idx_wait(1)
    gather_start(1, 1)

    def body(i, carry):
        base = 4 * i
        for k in range(4):
            p = k % 2
            gather_wait(p)
            pltpu.sync_copy(rows[p], acc.at[idxb.at[k, 1]], add=True)
            idx_start(k, jnp.minimum(base + k + 4, G - 1))
            k2 = (k + 2) % 4
            idx_wait(k2)
            gather_start(k2, p)
        return carry

    lax.fori_loop(0, G // 4, body, 0)
    idx_wait(2)
    idx_wait(3)
    gather_wait(0)
    gather_wait(1)
    plsc.subcore_barrier()
    pltpu.sync_copy(acc.at[pl.ds(r0, NPT)], out.at[c, pl.ds(r0, NPT)])


def _tc_norms_body(deg_ref, ns_ref, nd_ref):
    ds = deg_ref[0, 0, :, 0:1] + deg_ref[1, 0, :, 0:1]
    dd = deg_ref[0, 1, :, 0:1] + deg_ref[1, 1, :, 0:1]
    ns_ref[...] = lax.rsqrt(jnp.maximum(ds, 1.0))
    nd_ref[...] = lax.rsqrt(jnp.maximum(dd, 1.0))


def _tc_pre_body(x_ref, ns_ref, w_ref, y_ref):
    ns = ns_ref[...][:N]
    y_ref[...] = jnp.dot(x_ref[...] * ns, w_ref[...],
                         preferred_element_type=jnp.float32)


def _tc_mid_body(p_ref, nd_ref, ns_ref, b_ref, w_ref, y_ref):
    h = (p_ref[0] + p_ref[1]) * nd_ref[...] + b_ref[...]
    h = jnp.maximum(h, 0.0) * ns_ref[...]
    y_ref[...] = jnp.dot(h, w_ref[...], preferred_element_type=jnp.float32)


def _tc_final_body(p_ref, nd_ref, b_ref, wout_ref, bout_ref, o_ref):
    nd = nd_ref[...][:N]
    h = (p_ref[0, :N] + p_ref[1, :N]) * nd + b_ref[...]
    m = jnp.sum(h, axis=0, keepdims=True) * (1.0 / N)
    o_ref[...] = jnp.dot(m, wout_ref[...],
                         preferred_element_type=jnp.float32) + bout_ref[...]


def _tc_norms(deg):
    return pl.pallas_call(
        _tc_norms_body,
        out_shape=(jax.ShapeDtypeStruct((N_pad, 1), jnp.float32),
                   jax.ShapeDtypeStruct((N_pad, 1), jnp.float32)),
    )(deg)


def _tc_pre(x, ns, W):
    return pl.pallas_call(
        _tc_pre_body,
        out_shape=jax.ShapeDtypeStruct((N, D), jnp.float32),
    )(x, ns, W)


def _tc_mid(p, nd, ns, b, W):
    return pl.pallas_call(
        _tc_mid_body,
        out_shape=jax.ShapeDtypeStruct((N_pad, D), jnp.float32),
    )(p, nd, ns, b, W)


def _tc_final(p, nd, b, Wout, bout):
    return pl.pallas_call(
        _tc_final_body,
        out_shape=jax.ShapeDtypeStruct((1, 1), jnp.float32),
    )(p, nd, b, Wout, bout)


def kernel(in_feat, edge_index, W1, b1, W2, b2, W3, b3, Wout, bout):
    src = edge_index[0]
    dst = edge_index[1]
    pad = E_pad - E
    sink = N + (jnp.arange(pad, dtype=jnp.int32) % (N_pad - N))
    spread = jnp.arange(pad, dtype=jnp.int32) % N
    src_a = jnp.concatenate([src, spread])
    dst_a = jnp.concatenate([dst, sink])
    src_d = jnp.concatenate([src, sink])
    ei = jnp.stack([src_a.reshape(NW, G, CH), dst_a.reshape(NW, G, CH)], axis=2)
    ei_d = jnp.stack(
        [src_d.reshape(NW, G, CH), dst_a.reshape(NW, G, CH)], axis=2
    ).reshape(NW, 2 * G, CH)

    ones_h = jnp.ones((CH, D), jnp.float32)
    zerosD = jnp.zeros((CH, D), jnp.float32)

    deg = _sc_degrees(ei_d, ones_h, zerosD)
    ns, nd = _tc_norms(deg)

    b1r = b1.reshape(1, D)
    b2r = b2.reshape(1, D)
    b3r = b3.reshape(1, D)

    y = _tc_pre(in_feat, ns, W1)
    p = _sc_aggregate(y, ei, zerosD)
    y = _tc_mid(p, nd, ns, b1r, W2)
    p = _sc_aggregate(y, ei, zerosD)
    y = _tc_mid(p, nd, ns, b2r, W2)
    p = _sc_aggregate(y, ei, zerosD)
    y = _tc_mid(p, nd, ns, b2r, W3)
    p = _sc_aggregate(y, ei, zerosD)
    out = _tc_final(p, nd, b3r, Wout, bout.reshape(1, 1))
    return out.reshape(1)
